# baseline (speedup 1.0000x reference)
import functools

import jax
import jax.numpy as jnp
from jax import lax
from jax.experimental import pallas as pl
from jax.experimental.pallas import tpu as pltpu
from jax.experimental.pallas import tpu_sc as plsc

N = 10000
D = 128
E = 320000

NC = 2
NS = 16
CHUNK = 64
NCHUNK = 160
EPW = NCHUNK * CHUNK
EPAD = NC * NS * EPW
NBUF = 4
NIDX = 8
NDUMMY = 16
ZR = 16
ROW_STEP = 624


def _sc_scatter_partials(feat, src_p, dst_p):
    mesh = plsc.VectorSubcoreMesh(core_axis_name="c", subcore_axis_name="s")

    @functools.partial(
        pl.kernel,
        out_type=jax.ShapeDtypeStruct((NC * N, D), jnp.float32),
        mesh=mesh,
        scratch_types=[
            pltpu.VMEM_SHARED((N + NDUMMY, D), jnp.float32),
            pltpu.VMEM((ZR, D), jnp.float32),
            pltpu.VMEM((NBUF, CHUNK, D), jnp.float32),
            *([pltpu.VMEM((CHUNK,), jnp.int32)] * NIDX),
            *([pltpu.VMEM((CHUNK,), jnp.int32)] * NIDX),
            *([pltpu.SemaphoreType.DMA] * NIDX),
            *([pltpu.SemaphoreType.DMA] * NBUF),
            *([pltpu.SemaphoreType.DMA] * NBUF),
        ],
    )
    def k(feat_hbm, src_hbm, dst_hbm, out_hbm, acc, zbuf, rows, *rest):
        srci = list(rest[0:NIDX])
        dsti = list(rest[NIDX:2 * NIDX])
        isem = list(rest[2 * NIDX:3 * NIDX])
        gsem = list(rest[3 * NIDX:3 * NIDX + NBUF])
        ssem = list(rest[3 * NIDX + NBUF:3 * NIDX + 2 * NBUF])
        c = lax.axis_index("c")
        s = lax.axis_index("s")
        w = c * NS + s
        base = w * EPW
        row0 = s * ROW_STEP
        ngran_z = jnp.where(s == NS - 1, (640 + NDUMMY) // ZR, ROW_STEP // ZR)
        ngran_w = jnp.where(s == NS - 1, 640 // ZR, ROW_STEP // ZR)

        def idx_start(m, sl):
            off = base + m * CHUNK
            pltpu.async_copy(src_hbm.at[pl.ds(off, CHUNK)], srci[sl], isem[sl])
            pltpu.async_copy(dst_hbm.at[pl.ds(off, CHUNK)], dsti[sl], isem[sl])

        def idx_wait(sl):
            pltpu.make_async_copy(src_hbm.at[pl.ds(0, CHUNK)], srci[sl], isem[sl]).wait()
            pltpu.make_async_copy(dst_hbm.at[pl.ds(0, CHUNK)], dsti[sl], isem[sl]).wait()

        def gather_start(sl, b):
            pltpu.async_copy(feat_hbm.at[srci[sl]], rows.at[b], gsem[b])

        def gather_wait(sl, b):
            pltpu.make_async_copy(feat_hbm.at[srci[sl]], rows.at[b], gsem[b]).wait()

        def scatter_start(sl, b):
            pltpu.async_copy(rows.at[b], acc.at[dsti[sl]], ssem[b], add=True)

        def scatter_drain(b):
            pltpu.make_async_copy(rows.at[b], acc.at[dsti[0]], ssem[b]).wait()

        for sl in range(4):
            idx_start(sl, sl)

        zv = jnp.zeros((16,), jnp.float32)

        def zb(i, carry):
            zbuf[i // (D // 16), pl.ds((i % (D // 16)) * 16, 16)] = zv
            return carry

        lax.fori_loop(0, ZR * (D // 16), zb, 0)

        def ib(j, carry):
            pltpu.sync_copy(zbuf, acc.at[pl.ds(row0 + j * ZR, ZR)])
            return carry

        lax.fori_loop(0, ngran_z, ib, 0)
        plsc.subcore_barrier()

        idx_wait(0)
        gather_start(0, 0)
        idx_wait(1)
        gather_start(1, 1)

        def outer(mm, carry):
            for b8 in range(NIDX):
                m = mm * NIDX + b8
                b = b8 % NBUF

                @pl.when((m >= 2) & (m + 2 < NCHUNK))
                def _():
                    scatter_drain((b8 + 2) % NBUF)

                @pl.when(m + 4 < NCHUNK)
                def _():
                    idx_start(m + 4, (b8 + 4) % NIDX)

                @pl.when(m + 2 < NCHUNK)
                def _():
                    idx_wait((b8 + 2) % NIDX)
                    gather_start((b8 + 2) % NIDX, (b8 + 2) % NBUF)

                gather_wait(b8, b)
                scatter_start(b8, b)
            return carry

        lax.fori_loop(0, NCHUNK // NIDX, outer, 0)
        for b in range(NBUF):
            scatter_drain(b)
        plsc.subcore_barrier()

        def wb(j, carry):
            pltpu.sync_copy(acc.at[pl.ds(row0 + j * ZR, ZR)],
                            out_hbm.at[pl.ds(c * N + row0 + j * ZR, ZR)])
            return carry

        lax.fori_loop(0, ngran_w, wb, 0)

    return k(feat, src_p, dst_p)


def _mlp(xin, partials, W1, b1, W2, b2, final_relu):
    R = 1000
    nblk = N // R

    def body(x_ref, p0_ref, p1_ref, w1_ref, b1_ref, w2_ref, b2_ref, o_ref):
        h = x_ref[...] + p0_ref[...] + p1_ref[...]
        h = jnp.dot(h, w1_ref[...], preferred_element_type=jnp.float32) + b1_ref[...]
        h = jnp.maximum(h, 0.0)
        o = jnp.dot(h, w2_ref[...], preferred_element_type=jnp.float32) + b2_ref[...]
        if final_relu:
            o = jnp.maximum(o, 0.0)
        o_ref[...] = o

    return pl.pallas_call(
        body,
        grid=(nblk,),
        in_specs=[
            pl.BlockSpec((R, D), lambda i: (i, 0)),
            pl.BlockSpec((R, D), lambda i: (i, 0)),
            pl.BlockSpec((R, D), lambda i: (i + nblk, 0)),
            pl.BlockSpec((D, D), lambda i: (0, 0)),
            pl.BlockSpec((1, D), lambda i: (0, 0)),
            pl.BlockSpec((D, D), lambda i: (0, 0)),
            pl.BlockSpec((1, D), lambda i: (0, 0)),
        ],
        out_specs=pl.BlockSpec((R, D), lambda i: (i, 0)),
        out_shape=jax.ShapeDtypeStruct((N, D), jnp.float32),
    )(xin, partials, partials, W1, b1.reshape(1, D), W2, b2.reshape(1, D))


def kernel(x, edge_index, W1_0, b1_0, W2_0, b2_0, W1_1, b1_1, W2_1, b2_1):
    src = edge_index[0].astype(jnp.int32)
    dst = edge_index[1].astype(jnp.int32)
    pad = EPAD - E
    src_p = jnp.concatenate([src, jnp.zeros((pad,), jnp.int32)])
    dst_p = jnp.concatenate([dst, jnp.full((pad,), N, jnp.int32)])
    p = _sc_scatter_partials(x, src_p, dst_p)
    h = _mlp(x, p, W1_0, b1_0, W2_0, b2_0, final_relu=True)
    p = _sc_scatter_partials(h, src_p, dst_p)
    return _mlp(h, p, W1_1, b1_1, W2_1, b2_1, final_relu=False)
